# Initial kernel scaffold; baseline (speedup 1.0000x reference)
import jax
import jax.numpy as jnp
from jax.experimental import pallas as pl


def kernel(cat_base_ixs, cat_ante_ixs, hvb_idx, hvb_val, hva_idx, hva_val, hvb_top, hva_top, worddists, sqworddists, corefons, use_gpu, ablate_sem, cat_embeds, hvec_embeds, fc1_w, fc1_b, fc2_w, fc2_b):
    raise NotImplementedError("write your pallas kernel here")


# speedup vs baseline: 6.8046x; 6.8046x over previous
import functools

import jax
import jax.numpy as jnp
from jax import lax
from jax.experimental import pallas as pl
from jax.experimental.pallas import tpu as pltpu
from jax.experimental.pallas import tpu_sc as plsc

B = 16384
SYN = 32
SEM = 64
HID = 128
OUT = 2
NNZ = 20

NC = 2
NS = 16
NW = NC * NS
LANES = 16

ROWS_PER_W = B // NW
CHUNK = 32
N_CHUNKS = ROWS_PER_W // CHUNK
G = CHUNK * NNZ
GSLICE = 80
NG = G // GSLICE


def _sc_body(hvb_idx2, hvb_val, hva_idx2, hva_val, hvb_top, hva_top,
             catb_ix, cata_ix, cat_tab, hv_tab,
             catb_out, cata_out, hvb_out, hva_out,
             idx_v, val_v, rows_v, top_v, acc_v, cidx_v, crows_v,
             sem, csem):
  wid = lax.axis_index("s") * NC + lax.axis_index("c")

  def do_chunk(ch, _):
    rbase = pl.multiple_of(wid * ROWS_PER_W + ch * CHUNK, CHUNK)

    for cix, cout in ((catb_ix, catb_out), (cata_ix, cata_out)):
      pltpu.sync_copy(cix.at[pl.ds(rbase, CHUNK)], cidx_v)
      pltpu.async_copy(cat_tab.at[cidx_v], crows_v, csem).wait()
      pltpu.sync_copy(crows_v, cout.at[pl.ds(rbase, CHUNK)])

    for idx2, val_f, top, out in ((hvb_idx2, hvb_val, hvb_top, hvb_out),
                                  (hva_idx2, hva_val, hva_top, hva_out)):
      i0 = pl.multiple_of(rbase * NNZ // GSLICE, NG)
      pltpu.sync_copy(idx2.at[pl.ds(i0, NG)], idx_v)
      pltpu.sync_copy(val_f.at[pl.ds(pl.multiple_of(rbase * NNZ, G), G)],
                      val_v.at[pl.ds(0, G)])
      pltpu.sync_copy(top.at[pl.ds(rbase, CHUNK)], top_v)
      for j in range(NG):
        pltpu.async_copy(hv_tab.at[idx_v.at[j]],
                         rows_v.at[pl.ds(j * GSLICE, GSLICE)], sem)
      for j in range(NG):
        pltpu.make_async_copy(hv_tab.at[idx_v.at[j]],
                              rows_v.at[pl.ds(j * GSLICE, GSLICE)], sem).wait()

      def do_row(b, _):
        accs = [top_v[b, pl.ds(k * LANES, LANES)] for k in range(SEM // LANES)]
        jb = b * NNZ
        vals0 = val_v[pl.ds(jb, LANES)]
        vals1 = val_v[pl.ds(jb + LANES, LANES)]
        for n in range(NNZ):
          jdx = jb + n
          w = vals0[n] if n < LANES else vals1[n - LANES]
          for k in range(SEM // LANES):
            accs[k] = accs[k] + w * rows_v[jdx, pl.ds(k * LANES, LANES)]
        for k in range(SEM // LANES):
          acc_v[b, pl.ds(k * LANES, LANES)] = accs[k]
        return _

      lax.fori_loop(0, CHUNK, do_row, 0)
      pltpu.sync_copy(acc_v, out.at[pl.ds(rbase, CHUNK)])
    return _

  lax.fori_loop(0, N_CHUNKS, do_chunk, 0)


def _sc_embed(hvb_idx2, hvb_val, hva_idx2, hva_val, hvb_top, hva_top,
              catb_ix, cata_ix, cat_tab, hv_tab):
  mesh = plsc.VectorSubcoreMesh(core_axis_name="c", subcore_axis_name="s")
  out_type = (
      jax.ShapeDtypeStruct((B, SYN), jnp.float32),
      jax.ShapeDtypeStruct((B, SYN), jnp.float32),
      jax.ShapeDtypeStruct((B, SEM), jnp.float32),
      jax.ShapeDtypeStruct((B, SEM), jnp.float32),
  )
  scratch = [
      pltpu.VMEM((NG, GSLICE), jnp.int32),
      pltpu.VMEM((G + LANES,), jnp.float32),
      pltpu.VMEM((G, SEM), jnp.float32),
      pltpu.VMEM((CHUNK, SEM), jnp.float32),
      pltpu.VMEM((CHUNK, SEM), jnp.float32),
      pltpu.VMEM((CHUNK,), jnp.int32),
      pltpu.VMEM((CHUNK, SYN), jnp.float32),
      pltpu.SemaphoreType.DMA,
      pltpu.SemaphoreType.DMA,
  ]
  return pl.kernel(_sc_body, out_type=out_type, mesh=mesh,
                   scratch_types=scratch,
                   compiler_params=pltpu.CompilerParams(
                       use_tc_tiling_on_sc=False))(
      hvb_idx2, hvb_val, hva_idx2, hva_val, hvb_top, hva_top,
      catb_ix, cata_ix, cat_tab, hv_tab)


def _mlp_body(catb, cata, hvbe, hvae, feats, w1b, w1a, w1hb, w1ha, w1f,
              b1, w2, b2, out):
  h = jnp.dot(catb[...], w1b[...], preferred_element_type=jnp.float32)
  h += jnp.dot(cata[...], w1a[...], preferred_element_type=jnp.float32)
  h += jnp.dot(hvbe[...], w1hb[...], preferred_element_type=jnp.float32)
  h += jnp.dot(hvae[...], w1ha[...], preferred_element_type=jnp.float32)
  h += jnp.dot(feats[...], w1f[...], preferred_element_type=jnp.float32)
  h += b1[...]
  h = jnp.maximum(h, 0.0)
  logits = jnp.dot(h, w2[...], preferred_element_type=jnp.float32) + b2[...]
  m = jnp.max(logits, axis=1, keepdims=True)
  e = logits - m
  out[...] = e - jnp.log(jnp.sum(jnp.exp(e), axis=1, keepdims=True))


def _mlp(catb, cata, hvbe, hvae, feats, w1b, w1a, w1hb, w1ha, w1f, b1, w2, b2):
  R = 2048
  grid = (B // R,)
  full = lambda shape: pl.BlockSpec(shape, lambda i: (0, 0))
  return pl.pallas_call(
      _mlp_body,
      grid=grid,
      in_specs=[
          pl.BlockSpec((R, SYN), lambda i: (i, 0)),
          pl.BlockSpec((R, SYN), lambda i: (i, 0)),
          pl.BlockSpec((R, SEM), lambda i: (i, 0)),
          pl.BlockSpec((R, SEM), lambda i: (i, 0)),
          pl.BlockSpec((R, 8), lambda i: (i, 0)),
          full((SYN, HID)), full((SYN, HID)), full((SEM, HID)),
          full((SEM, HID)), full((8, HID)), full((1, HID)),
          full((HID, OUT)), full((1, OUT)),
      ],
      out_specs=pl.BlockSpec((R, OUT), lambda i: (i, 0)),
      out_shape=jax.ShapeDtypeStruct((B, OUT), jnp.float32),
  )(catb, cata, hvbe, hvae, feats, w1b, w1a, w1hb, w1ha, w1f, b1, w2, b2)


def kernel(cat_base_ixs, cat_ante_ixs, hvb_idx, hvb_val, hva_idx, hva_val,
           hvb_top, hva_top, worddists, sqworddists, corefons,
           use_gpu, ablate_sem,
           cat_embeds, hvec_embeds, fc1_w, fc1_b, fc2_w, fc2_b):
  catb_ix = cat_base_ixs.astype(jnp.int32)
  cata_ix = cat_ante_ixs.astype(jnp.int32)
  hvb_idx2 = hvb_idx.astype(jnp.int32).reshape(B * NNZ // GSLICE, GSLICE)
  hva_idx2 = hva_idx.astype(jnp.int32).reshape(B * NNZ // GSLICE, GSLICE)
  hvb_valf = hvb_val.reshape(B * NNZ)
  hva_valf = hva_val.reshape(B * NNZ)

  catb_e, cata_e, hvb_e, hva_e = _sc_embed(
      hvb_idx2, hvb_valf, hva_idx2, hva_valf, hvb_top, hva_top,
      catb_ix, cata_ix, cat_embeds, hvec_embeds)

  feats = jnp.zeros((B, 8), jnp.float32)
  feats = feats.at[:, 0].set(worddists)
  feats = feats.at[:, 1].set(sqworddists)
  feats = feats.at[:, 2].set(corefons)

  w1 = fc1_w.T
  w1b = w1[:SYN]
  w1a = w1[SYN:2 * SYN]
  w1hb = w1[2 * SYN:2 * SYN + SEM]
  w1ha = w1[2 * SYN + SEM:2 * SYN + 2 * SEM]
  w1f = jnp.zeros((8, HID), jnp.float32).at[:3].set(w1[2 * SYN + 2 * SEM:])
  b1 = fc1_b.reshape(1, HID)
  w2 = fc2_w.T
  b2 = fc2_b.reshape(1, OUT)

  return _mlp(catb_e, cata_e, hvb_e, hva_e, feats,
              w1b, w1a, w1hb, w1ha, w1f, b1, w2, b2)
